# baseline (speedup 1.0000x reference)
import jax
import jax.numpy as jnp
from jax.experimental import pallas as pl
from jax.experimental.pallas import tpu as pltpu

B, T, D = 32, 576, 256
N_SUB, K, SUB = 8, 256, 32
HIDDEN = 1024
CLASSES = 1000
TM = 1024
HALF = 512
NK = N_SUB * K


def _body(z_ref, cbtbd_ref, csq_ref, cbs_ref, w1_ref, b1_ref, w2_ref, b2_ref,
          out_ref, onehot_scr):
    iota = jax.lax.broadcasted_iota(jnp.int32, (HALF, K), 1)
    for h0 in range(0, TM, HALF):
        rows = pl.ds(h0, HALF)
        zt = z_ref[rows, :]
        for n in range(N_SUB):
            dist = jnp.dot(zt, cbtbd_ref[:, n * K:(n + 1) * K],
                           preferred_element_type=jnp.float32) \
                + csq_ref[:, n * K:(n + 1) * K]
            code = jnp.argmin(dist, axis=1)
            onehot_scr[rows, n * K:(n + 1) * K] = (
                iota == code[:, None]).astype(jnp.bfloat16)
        q = jnp.dot(onehot_scr[rows, :], cbs_ref[...],
                    preferred_element_type=jnp.float32)
        h = jnp.maximum(jnp.dot(q.astype(jnp.bfloat16), w1_ref[...],
                                preferred_element_type=jnp.float32)
                        + b1_ref[...], 0.0)
        out_ref[rows, :] = jnp.dot(h.astype(jnp.bfloat16), w2_ref[...],
                                   preferred_element_type=jnp.float32) \
            + b2_ref[...]


@jax.jit
def kernel(z, codebook, W1, b1, W2, b2):
    b, t, d = z.shape
    n_sub, k, sub = codebook.shape
    bt = b * t
    z2 = z.reshape(bt, d)
    cbT = codebook.transpose(0, 2, 1)
    cbtbd = jnp.concatenate(
        [jnp.pad(-2.0 * cbT[n], ((0, 0), (n * k, (n_sub - 1 - n) * k)))
         for n in range(n_sub)], axis=0)
    csq = jnp.sum(codebook * codebook, axis=-1).reshape(1, n_sub * k)
    cbs = jnp.concatenate(
        [jnp.pad(codebook[n], ((0, 0), (n * sub, d - (n + 1) * sub)))
         for n in range(n_sub)], axis=0).astype(jnp.bfloat16)
    W1b = W1.astype(jnp.bfloat16)
    W2b = W2.astype(jnp.bfloat16)
    b2r = b2.reshape(1, CLASSES)
    b1r = b1.reshape(1, HIDDEN)

    grid = (bt // TM,)
    out = pl.pallas_call(
        _body,
        grid=grid,
        in_specs=[
            pl.BlockSpec((TM, d), lambda i: (i, 0)),
            pl.BlockSpec((d, NK), lambda i: (0, 0)),
            pl.BlockSpec((1, NK), lambda i: (0, 0)),
            pl.BlockSpec((NK, d), lambda i: (0, 0)),
            pl.BlockSpec((d, HIDDEN), lambda i: (0, 0)),
            pl.BlockSpec((1, HIDDEN), lambda i: (0, 0)),
            pl.BlockSpec((HIDDEN, CLASSES), lambda i: (0, 0)),
            pl.BlockSpec((1, CLASSES), lambda i: (0, 0)),
        ],
        out_specs=pl.BlockSpec((TM, CLASSES), lambda i: (i, 0)),
        out_shape=jax.ShapeDtypeStruct((bt, CLASSES), jnp.float32),
        scratch_shapes=[pltpu.VMEM((TM, NK), jnp.bfloat16)],
        compiler_params=pltpu.CompilerParams(
            dimension_semantics=("arbitrary",),
        ),
    )(z2, cbtbd, csq, cbs, W1b, b1r, W2b, b2r)
    return out.reshape(b, t, CLASSES)


# speedup vs baseline: 1.0929x; 1.0929x over previous
import jax
import jax.numpy as jnp
from jax.experimental import pallas as pl
from jax.experimental.pallas import tpu as pltpu

B, T, D = 32, 576, 256
N_SUB, K, SUB = 8, 256, 32
HIDDEN = 1024
CLASSES = 1000
TM = 2048
HALF = 1024
NK = N_SUB * K


def _body(z_ref, cbtbd_ref, csq_ref, cbs_ref, w1_ref, b1_ref, w2_ref, b2_ref,
          out_ref, onehot_scr):
    iota = jax.lax.broadcasted_iota(jnp.int32, (HALF, K), 1)
    for h0 in range(0, TM, HALF):
        rows = pl.ds(h0, HALF)
        zt = z_ref[rows, :]
        for n in range(N_SUB):
            dist = jnp.dot(zt, cbtbd_ref[:, n * K:(n + 1) * K],
                           preferred_element_type=jnp.float32) \
                + csq_ref[:, n * K:(n + 1) * K]
            code = jnp.argmin(dist, axis=1)
            onehot_scr[rows, n * K:(n + 1) * K] = (
                iota == code[:, None]).astype(jnp.bfloat16)
        q = jnp.dot(onehot_scr[rows, :], cbs_ref[...],
                    preferred_element_type=jnp.float32)
        h = jnp.maximum(jnp.dot(q.astype(jnp.bfloat16), w1_ref[...],
                                preferred_element_type=jnp.float32)
                        + b1_ref[...], 0.0)
        out_ref[rows, :] = jnp.dot(h.astype(jnp.bfloat16), w2_ref[...],
                                   preferred_element_type=jnp.float32) \
            + b2_ref[...]


@jax.jit
def kernel(z, codebook, W1, b1, W2, b2):
    b, t, d = z.shape
    n_sub, k, sub = codebook.shape
    bt = b * t
    z2 = z.reshape(bt, d)
    cbT = codebook.transpose(0, 2, 1)
    cbtbd = jnp.concatenate(
        [jnp.pad(-2.0 * cbT[n], ((0, 0), (n * k, (n_sub - 1 - n) * k)))
         for n in range(n_sub)], axis=0)
    csq = jnp.sum(codebook * codebook, axis=-1).reshape(1, n_sub * k)
    cbs = jnp.concatenate(
        [jnp.pad(codebook[n], ((0, 0), (n * sub, d - (n + 1) * sub)))
         for n in range(n_sub)], axis=0).astype(jnp.bfloat16)
    W1b = W1.astype(jnp.bfloat16)
    W2b = W2.astype(jnp.bfloat16)
    b2r = b2.reshape(1, CLASSES)
    b1r = b1.reshape(1, HIDDEN)

    grid = (bt // TM,)
    out = pl.pallas_call(
        _body,
        grid=grid,
        in_specs=[
            pl.BlockSpec((TM, d), lambda i: (i, 0)),
            pl.BlockSpec((d, NK), lambda i: (0, 0)),
            pl.BlockSpec((1, NK), lambda i: (0, 0)),
            pl.BlockSpec((NK, d), lambda i: (0, 0)),
            pl.BlockSpec((d, HIDDEN), lambda i: (0, 0)),
            pl.BlockSpec((1, HIDDEN), lambda i: (0, 0)),
            pl.BlockSpec((HIDDEN, CLASSES), lambda i: (0, 0)),
            pl.BlockSpec((1, CLASSES), lambda i: (0, 0)),
        ],
        out_specs=pl.BlockSpec((TM, CLASSES), lambda i: (i, 0)),
        out_shape=jax.ShapeDtypeStruct((bt, CLASSES), jnp.float32),
        scratch_shapes=[pltpu.VMEM((TM, NK), jnp.bfloat16)],
        compiler_params=pltpu.CompilerParams(
            dimension_semantics=("arbitrary",),
        ),
    )(z2, cbtbd, csq, cbs, W1b, b1r, W2b, b2r)
    return out.reshape(b, t, CLASSES)


# speedup vs baseline: 1.2857x; 1.1764x over previous
import jax
import jax.numpy as jnp
from jax.experimental import pallas as pl
from jax.experimental.pallas import tpu as pltpu

B, T, D = 32, 576, 256
N_SUB, K, SUB = 8, 256, 32
HIDDEN = 1024
CLASSES = 1000
TM = 2048
HALF = 1024
NK = N_SUB * K


def _body(z_ref, cbtbd_ref, csq_ref, cbs_ref, w1_ref, b1_ref, w2_ref, b2_ref,
          out_ref, onehot_scr):
    iota = jax.lax.broadcasted_iota(jnp.int32, (HALF, K), 1)
    for h0 in range(0, TM, HALF):
        rows = pl.ds(h0, HALF)
        zt = z_ref[rows, :]
        for n in range(N_SUB):
            dist = jnp.dot(zt, cbtbd_ref[:, n * K:(n + 1) * K],
                           preferred_element_type=jnp.float32) \
                + csq_ref[:, n * K:(n + 1) * K]
            m = jnp.min(dist, axis=1, keepdims=True)
            code = jnp.min(jnp.where(dist == m, iota, K), axis=1,
                           keepdims=True)
            onehot_scr[rows, n * K:(n + 1) * K] = (
                iota == code).astype(jnp.bfloat16)
        q = jnp.dot(onehot_scr[rows, :], cbs_ref[...],
                    preferred_element_type=jnp.float32)
        h = jnp.maximum(jnp.dot(q.astype(jnp.bfloat16), w1_ref[...],
                                preferred_element_type=jnp.float32)
                        + b1_ref[...], 0.0)
        out_ref[rows, :] = jnp.dot(h.astype(jnp.bfloat16), w2_ref[...],
                                   preferred_element_type=jnp.float32) \
            + b2_ref[...]


@jax.jit
def kernel(z, codebook, W1, b1, W2, b2):
    b, t, d = z.shape
    n_sub, k, sub = codebook.shape
    bt = b * t
    z2 = z.reshape(bt, d)
    cbT = codebook.transpose(0, 2, 1)
    cbtbd = jnp.concatenate(
        [jnp.pad(-2.0 * cbT[n], ((0, 0), (n * k, (n_sub - 1 - n) * k)))
         for n in range(n_sub)], axis=0)
    csq = jnp.sum(codebook * codebook, axis=-1).reshape(1, n_sub * k)
    cbs = jnp.concatenate(
        [jnp.pad(codebook[n], ((0, 0), (n * sub, d - (n + 1) * sub)))
         for n in range(n_sub)], axis=0).astype(jnp.bfloat16)
    W1b = W1.astype(jnp.bfloat16)
    W2b = W2.astype(jnp.bfloat16)
    b2r = b2.reshape(1, CLASSES)
    b1r = b1.reshape(1, HIDDEN)

    grid = (bt // TM,)
    out = pl.pallas_call(
        _body,
        grid=grid,
        in_specs=[
            pl.BlockSpec((TM, d), lambda i: (i, 0)),
            pl.BlockSpec((d, NK), lambda i: (0, 0)),
            pl.BlockSpec((1, NK), lambda i: (0, 0)),
            pl.BlockSpec((NK, d), lambda i: (0, 0)),
            pl.BlockSpec((d, HIDDEN), lambda i: (0, 0)),
            pl.BlockSpec((1, HIDDEN), lambda i: (0, 0)),
            pl.BlockSpec((HIDDEN, CLASSES), lambda i: (0, 0)),
            pl.BlockSpec((1, CLASSES), lambda i: (0, 0)),
        ],
        out_specs=pl.BlockSpec((TM, CLASSES), lambda i: (i, 0)),
        out_shape=jax.ShapeDtypeStruct((bt, CLASSES), jnp.float32),
        scratch_shapes=[pltpu.VMEM((TM, NK), jnp.bfloat16)],
        compiler_params=pltpu.CompilerParams(
            dimension_semantics=("arbitrary",),
        ),
    )(z2, cbtbd, csq, cbs, W1b, b1r, W2b, b2r)
    return out.reshape(b, t, CLASSES)


# speedup vs baseline: 1.3091x; 1.0182x over previous
import jax
import jax.numpy as jnp
from jax.experimental import pallas as pl
from jax.experimental.pallas import tpu as pltpu

B, T, D = 32, 576, 256
N_SUB, K, SUB = 8, 256, 32
HIDDEN = 1024
CLASSES = 1000
TM = 3072
HALF = 1024
NK = N_SUB * K


def _body(z_ref, cbtbd_ref, csq_ref, cbs_ref, w1_ref, b1_ref, w2_ref, b2_ref,
          out_ref, onehot_scr):
    iota = jax.lax.broadcasted_iota(jnp.int32, (HALF, K), 1)
    for h0 in range(0, TM, HALF):
        rows = pl.ds(h0, HALF)
        zt = z_ref[rows, :]
        for n in range(N_SUB):
            dist = jnp.dot(zt, cbtbd_ref[:, n * K:(n + 1) * K],
                           preferred_element_type=jnp.float32) \
                + csq_ref[:, n * K:(n + 1) * K]
            m = jnp.min(dist, axis=1, keepdims=True)
            code = jnp.min(jnp.where(dist == m, iota, K), axis=1,
                           keepdims=True)
            onehot_scr[rows, n * K:(n + 1) * K] = (
                iota == code).astype(jnp.bfloat16)
        q = jnp.dot(onehot_scr[rows, :], cbs_ref[...],
                    preferred_element_type=jnp.float32)
        h = jnp.maximum(jnp.dot(q.astype(jnp.bfloat16), w1_ref[...],
                                preferred_element_type=jnp.float32)
                        + b1_ref[...], 0.0)
        out_ref[rows, :] = jnp.dot(h.astype(jnp.bfloat16), w2_ref[...],
                                   preferred_element_type=jnp.float32) \
            + b2_ref[...]


@jax.jit
def kernel(z, codebook, W1, b1, W2, b2):
    b, t, d = z.shape
    n_sub, k, sub = codebook.shape
    bt = b * t
    z2 = z.reshape(bt, d)
    cbT = codebook.transpose(0, 2, 1)
    cbtbd = jnp.concatenate(
        [jnp.pad(-2.0 * cbT[n], ((0, 0), (n * k, (n_sub - 1 - n) * k)))
         for n in range(n_sub)], axis=0)
    csq = jnp.sum(codebook * codebook, axis=-1).reshape(1, n_sub * k)
    cbs = jnp.concatenate(
        [jnp.pad(codebook[n], ((0, 0), (n * sub, d - (n + 1) * sub)))
         for n in range(n_sub)], axis=0).astype(jnp.bfloat16)
    W1b = W1.astype(jnp.bfloat16)
    W2b = W2.astype(jnp.bfloat16)
    b2r = b2.reshape(1, CLASSES)
    b1r = b1.reshape(1, HIDDEN)

    grid = (bt // TM,)
    out = pl.pallas_call(
        _body,
        grid=grid,
        in_specs=[
            pl.BlockSpec((TM, d), lambda i: (i, 0)),
            pl.BlockSpec((d, NK), lambda i: (0, 0)),
            pl.BlockSpec((1, NK), lambda i: (0, 0)),
            pl.BlockSpec((NK, d), lambda i: (0, 0)),
            pl.BlockSpec((d, HIDDEN), lambda i: (0, 0)),
            pl.BlockSpec((1, HIDDEN), lambda i: (0, 0)),
            pl.BlockSpec((HIDDEN, CLASSES), lambda i: (0, 0)),
            pl.BlockSpec((1, CLASSES), lambda i: (0, 0)),
        ],
        out_specs=pl.BlockSpec((TM, CLASSES), lambda i: (i, 0)),
        out_shape=jax.ShapeDtypeStruct((bt, CLASSES), jnp.float32),
        scratch_shapes=[pltpu.VMEM((TM, NK), jnp.bfloat16)],
        compiler_params=pltpu.CompilerParams(
            dimension_semantics=("arbitrary",),
        ),
    )(z2, cbtbd, csq, cbs, W1b, b1r, W2b, b2r)
    return out.reshape(b, t, CLASSES)
